# baseline (speedup 1.0000x reference)
import functools

import jax
import jax.numpy as jnp
from jax import lax
from jax.experimental import pallas as pl
from jax.experimental.pallas import tpu as pltpu
from jax.experimental.pallas import tpu_sc as plsc

_NC = 2
_NS = 16
_NW = _NC * _NS
_DS = 8192

_MESH = plsc.VectorSubcoreMesh(core_axis_name="c", subcore_axis_name="s")
_CP = pltpu.CompilerParams(use_tc_tiling_on_sc=False)


def _perm_idx(v):
    s = _DS
    lg = s.bit_length() - 1
    return ((v & jnp.int32(-2 * s))
            | ((v & jnp.int32(s - 1)) << 1)
            | ((v >> lg) & jnp.int32(1)))


def _detile_body(t0, t1, tout):
    tout[:, 0:64] = jnp.transpose(t0[...])
    tout[:, 64:128] = jnp.transpose(t1[...])


def _detile(t):
    V, H = t.shape
    S = _DS
    assert H == 64
    grid = (V + 2 * S - 1) // (2 * S)
    last = (V + S - 1) // S - 1
    out = pl.pallas_call(
        _detile_body,
        grid=(grid,),
        in_specs=[
            pl.BlockSpec((H, S), lambda j: (0, jnp.minimum(2 * j, last))),
            pl.BlockSpec((H, S), lambda j: (0, jnp.minimum(2 * j + 1, last))),
        ],
        out_specs=pl.BlockSpec((S, 2 * H), lambda j: (j, 0)),
        out_shape=jax.ShapeDtypeStruct((grid * S, 2 * H), jnp.float32),
    )(t.T, t.T)
    return out.reshape(grid * 2 * S, H)


def _sc_small_gathers(user, cate, u_emb, c_emb):
    B = user.shape[0]
    H = u_emb.shape[1]
    nb = B // _NW
    f32 = jnp.float32
    out_t = jax.ShapeDtypeStruct((B, H), f32)

    @functools.partial(
        pl.kernel,
        mesh=_MESH,
        out_type=[out_t, out_t],
        compiler_params=_CP,
        scratch_types=[
            pltpu.VMEM((nb, H), f32),
            pltpu.VMEM((nb, H), f32),
            pltpu.VMEM((nb,), jnp.int32),
            pltpu.VMEM((nb,), jnp.int32),
            pltpu.SemaphoreType.DMA,
            pltpu.SemaphoreType.DMA,
        ],
    )
    def sc_a(user_h, cate_h, uemb_h, cemb_h, u_out, c_out,
             ubuf, cbuf, uidx, cidx, semu, semc):
        wid = lax.axis_index("s") * _NC + lax.axis_index("c")
        b0 = wid * nb
        pltpu.sync_copy(user_h.at[pl.ds(b0, nb)], uidx)
        pltpu.sync_copy(cate_h.at[pl.ds(b0, nb)], cidx)
        for ch in range(nb // 16):
            s = pl.ds(ch * 16, 16)
            uidx[s] = _perm_idx(uidx[s])
            cidx[s] = _perm_idx(cidx[s])
        pltpu.async_copy(uemb_h.at[uidx], ubuf, semu)
        pltpu.async_copy(cemb_h.at[cidx], cbuf, semc)
        pltpu.make_async_copy(uemb_h.at[uidx], ubuf, semu).wait()
        pltpu.sync_copy(ubuf, u_out.at[pl.ds(b0, nb)])
        pltpu.make_async_copy(cemb_h.at[cidx], cbuf, semc).wait()
        pltpu.sync_copy(cbuf, c_out.at[pl.ds(b0, nb)])

    return sc_a(user, cate, u_emb, c_emb)


def _sc_pool(item, hist, i_emb):
    L, B = hist.shape
    H = i_emb.shape[1]
    nb = B // _NW
    assert B % (8 * _NW) == 0 and L % 2 == 0 and L >= 6 and nb <= 128
    assert H % 16 == 0
    f32 = jnp.float32
    out_t = jax.ShapeDtypeStruct((B, H), f32)

    @functools.partial(
        pl.kernel,
        mesh=_MESH,
        out_type=[out_t, out_t],
        compiler_params=_CP,
        scratch_types=[
            pltpu.VMEM((L, nb), jnp.int32),
            pltpu.VMEM_SHARED((B // _NC, H), f32),
            pltpu.VMEM((nb, H), f32),
            pltpu.VMEM((nb, H), f32),
            pltpu.VMEM((nb, H), f32),
            pltpu.VMEM((nb, H), f32),
            pltpu.VMEM((nb, H), f32),
            pltpu.VMEM((nb, H), f32),
            pltpu.VMEM((nb, H), f32),
            pltpu.VMEM((nb, H), f32),
            pltpu.VMEM((nb, H), f32),
            pltpu.VMEM((nb,), jnp.int32),
            pltpu.VMEM((nb,), jnp.int32),
            pltpu.SemaphoreType.DMA,
            pltpu.SemaphoreType.DMA,
            pltpu.SemaphoreType.DMA,
            pltpu.SemaphoreType.DMA,
            pltpu.SemaphoreType.DMA,
            pltpu.SemaphoreType.DMA,
            pltpu.SemaphoreType.DMA,
            pltpu.SemaphoreType.DMA,
            pltpu.SemaphoreType.DMA,
        ],
    )
    def sc_b(item_h, hist_h, iemb_h, it_out, cur_out,
             hist_v, accum, buf0, buf1, buf2, buf3, buf4, buf5, buf6, buf7,
             ibuf, iidx, lidx,
             sem0, sem1, sem2, sem3, sem4, sem5, sem6, sem7, semi):
        sid = lax.axis_index("s")
        wid = sid * _NC + lax.axis_index("c")
        b0 = wid * nb
        a0 = sid * nb

        pltpu.sync_copy(hist_h.at[:, pl.ds(b0, nb)], hist_v)
        pltpu.sync_copy(item_h.at[pl.ds(b0, nb)], iidx)

        @pl.loop(0, L)
        def _(l):
            for ch in range(nb // 16):
                s = pl.ds(ch * 16, 16)
                hist_v[l, s] = _perm_idx(hist_v[l, s])

        for ch in range(nb // 16):
            s = pl.ds(ch * 16, 16)
            iidx[s] = _perm_idx(iidx[s])

        pltpu.async_copy(iemb_h.at[iidx], ibuf, semi)

        for j in range(nb // 16):
            lidx[pl.ds(j * 16, 16)] = lax.iota(jnp.int32, 16) + j * 16 + a0

        @pl.loop(0, nb)
        def _(i):
            for ch in range(H // 16):
                buf0[i, pl.ds(ch * 16, 16)] = jnp.zeros((16,), f32)

        pltpu.sync_copy(buf0, accum.at[pl.ds(a0, nb)])

        nring = 8
        bufs = (buf0, buf1, buf2, buf3, buf4, buf5, buf6, buf7)
        sems = (sem0, sem1, sem2, sem3, sem4, sem5, sem6, sem7)
        assert L % nring == 0 and L >= 2 * nring
        for k in range(nring):
            pltpu.async_copy(iemb_h.at[hist_v.at[k]], bufs[k], sems[k])

        @pl.loop(0, L - nring, step=nring)
        def _(l):
            for k in range(nring):
                pltpu.make_async_copy(
                    iemb_h.at[hist_v.at[l + k]], bufs[k], sems[k]).wait()
                pltpu.sync_copy(bufs[k], accum.at[lidx], add=True)
                pltpu.async_copy(
                    iemb_h.at[hist_v.at[l + nring + k]], bufs[k], sems[k])

        for k in range(nring):
            pltpu.make_async_copy(
                iemb_h.at[hist_v.at[L - nring + k]], bufs[k], sems[k]).wait()
            pltpu.sync_copy(bufs[k], accum.at[lidx], add=True)

        pltpu.make_async_copy(iemb_h.at[iidx], ibuf, semi).wait()
        pltpu.sync_copy(ibuf, it_out.at[pl.ds(b0, nb)])
        pltpu.sync_copy(accum.at[pl.ds(a0, nb)], cur_out.at[pl.ds(b0, nb)])

    return sc_b(item, hist, i_emb)


def _dice(x, alpha, eps=1e-8):
    mean = jnp.mean(x, axis=0, keepdims=True)
    var = jnp.mean((x - mean) ** 2, axis=0, keepdims=True)
    x_norm = (x - mean) / jnp.sqrt(var + eps)
    p = jax.nn.sigmoid(x_norm)
    return p * x + (1.0 - p) * alpha * x


def _mlp_body(u, it, c, cur, W1, b1, a1, W2, b2, a2, W3, b3, o):
    H = u.shape[1]
    f32 = jnp.float32
    x = jnp.dot(u[...], W1[0:H, :], preferred_element_type=f32)
    x = x + jnp.dot(it[...], W1[H:2 * H, :], preferred_element_type=f32)
    x = x + jnp.dot(c[...], W1[2 * H:3 * H, :], preferred_element_type=f32)
    x = x + jnp.dot(cur[...], W1[3 * H:4 * H, :], preferred_element_type=f32)
    x = x + b1[...]
    x = _dice(x, a1[...])
    x = jnp.dot(x, W2[...], preferred_element_type=f32) + b2[...]
    x = _dice(x, a2[...])
    o[...] = jnp.dot(x, W3[...], preferred_element_type=f32) + b3[...]


def kernel(user, hist, item, cate, u_emb, i_emb, c_emb, W1, b1, a1, W2, b2, a2, W3, b3):
    B = user.shape[0]
    u_emb = _detile(u_emb)
    c_emb = _detile(c_emb)
    u, c = _sc_small_gathers(user, cate, u_emb, c_emb)
    i_emb = _detile(i_emb)
    it, cur = _sc_pool(item, hist, i_emb)
    out = pl.pallas_call(
        _mlp_body,
        out_shape=jax.ShapeDtypeStruct((B, W3.shape[1]), jnp.float32),
    )(u, it, c, cur,
      W1, b1.reshape(1, -1), a1.reshape(1, -1),
      W2, b2.reshape(1, -1), a2.reshape(1, -1),
      W3, b3.reshape(1, -1))
    return out


# speedup vs baseline: 1.1797x; 1.1797x over previous
import functools

import jax
import jax.numpy as jnp
from jax import lax
from jax.experimental import pallas as pl
from jax.experimental.pallas import tpu as pltpu
from jax.experimental.pallas import tpu_sc as plsc

_NC = 2
_NS = 16
_NW = _NC * _NS
_DS = 8192

_MESH = plsc.VectorSubcoreMesh(core_axis_name="c", subcore_axis_name="s")
_CP = pltpu.CompilerParams(use_tc_tiling_on_sc=False)


def _perm_idx(v):
    s = _DS
    lg = s.bit_length() - 1
    return ((v & jnp.int32(-2 * s))
            | ((v & jnp.int32(s - 1)) << 1)
            | ((v >> lg) & jnp.int32(1)))


def _detile_body(t0, t1, tout):
    x = jnp.concatenate([t0[...], t1[...]], axis=0)
    tout[...] = jnp.transpose(x)


def _detile(t):
    V, H = t.shape
    S = _DS
    assert H == 64
    grid = (V + 2 * S - 1) // (2 * S)
    last = (V + S - 1) // S - 1
    out = pl.pallas_call(
        _detile_body,
        grid=(grid,),
        in_specs=[
            pl.BlockSpec((H, S), lambda j: (0, jnp.minimum(2 * j, last))),
            pl.BlockSpec((H, S), lambda j: (0, jnp.minimum(2 * j + 1, last))),
        ],
        out_specs=pl.BlockSpec((S, 2 * H), lambda j: (j, 0)),
        out_shape=jax.ShapeDtypeStruct((grid * S, 2 * H), jnp.float32),
    )(t.T, t.T)
    return out.reshape(grid * 2 * S, H)


def _sc_small_gathers(user, cate, u_emb, c_emb):
    B = user.shape[0]
    H = u_emb.shape[1]
    nb = B // _NW
    f32 = jnp.float32
    out_t = jax.ShapeDtypeStruct((B, H), f32)

    @functools.partial(
        pl.kernel,
        mesh=_MESH,
        out_type=[out_t, out_t],
        compiler_params=_CP,
        scratch_types=[
            pltpu.VMEM((nb, H), f32),
            pltpu.VMEM((nb, H), f32),
            pltpu.VMEM((nb,), jnp.int32),
            pltpu.VMEM((nb,), jnp.int32),
            pltpu.SemaphoreType.DMA,
            pltpu.SemaphoreType.DMA,
        ],
    )
    def sc_a(user_h, cate_h, uemb_h, cemb_h, u_out, c_out,
             ubuf, cbuf, uidx, cidx, semu, semc):
        wid = lax.axis_index("s") * _NC + lax.axis_index("c")
        b0 = wid * nb
        pltpu.sync_copy(user_h.at[pl.ds(b0, nb)], uidx)
        pltpu.sync_copy(cate_h.at[pl.ds(b0, nb)], cidx)
        for ch in range(nb // 16):
            s = pl.ds(ch * 16, 16)
            uidx[s] = _perm_idx(uidx[s])
            cidx[s] = _perm_idx(cidx[s])
        pltpu.async_copy(uemb_h.at[uidx], ubuf, semu)
        pltpu.async_copy(cemb_h.at[cidx], cbuf, semc)
        pltpu.make_async_copy(uemb_h.at[uidx], ubuf, semu).wait()
        pltpu.sync_copy(ubuf, u_out.at[pl.ds(b0, nb)])
        pltpu.make_async_copy(cemb_h.at[cidx], cbuf, semc).wait()
        pltpu.sync_copy(cbuf, c_out.at[pl.ds(b0, nb)])

    return sc_a(user, cate, u_emb, c_emb)


def _sc_pool(item, hist, i_emb):
    L, B = hist.shape
    H = i_emb.shape[1]
    nb = B // _NW
    assert B % (8 * _NW) == 0 and L % 2 == 0 and L >= 6 and nb <= 128
    assert H % 16 == 0
    f32 = jnp.float32
    out_t = jax.ShapeDtypeStruct((B, H), f32)

    @functools.partial(
        pl.kernel,
        mesh=_MESH,
        out_type=[out_t, out_t],
        compiler_params=_CP,
        scratch_types=[
            pltpu.VMEM((L, nb), jnp.int32),
            pltpu.VMEM_SHARED((B // _NC, H), f32),
            pltpu.VMEM((nb, H), f32),
            pltpu.VMEM((nb, H), f32),
            pltpu.VMEM((nb, H), f32),
            pltpu.VMEM((nb, H), f32),
            pltpu.VMEM((nb, H), f32),
            pltpu.VMEM((nb, H), f32),
            pltpu.VMEM((nb, H), f32),
            pltpu.VMEM((nb, H), f32),
            pltpu.VMEM((nb, H), f32),
            pltpu.VMEM((nb,), jnp.int32),
            pltpu.VMEM((nb,), jnp.int32),
            pltpu.SemaphoreType.DMA,
            pltpu.SemaphoreType.DMA,
            pltpu.SemaphoreType.DMA,
            pltpu.SemaphoreType.DMA,
            pltpu.SemaphoreType.DMA,
            pltpu.SemaphoreType.DMA,
            pltpu.SemaphoreType.DMA,
            pltpu.SemaphoreType.DMA,
            pltpu.SemaphoreType.DMA,
        ],
    )
    def sc_b(item_h, hist_h, iemb_h, it_out, cur_out,
             hist_v, accum, buf0, buf1, buf2, buf3, buf4, buf5, buf6, buf7,
             ibuf, iidx, lidx,
             sem0, sem1, sem2, sem3, sem4, sem5, sem6, sem7, semi):
        sid = lax.axis_index("s")
        wid = sid * _NC + lax.axis_index("c")
        b0 = wid * nb
        a0 = sid * nb

        pltpu.sync_copy(hist_h.at[:, pl.ds(b0, nb)], hist_v)
        pltpu.sync_copy(item_h.at[pl.ds(b0, nb)], iidx)

        @pl.loop(0, L)
        def _(l):
            for ch in range(nb // 16):
                s = pl.ds(ch * 16, 16)
                hist_v[l, s] = _perm_idx(hist_v[l, s])

        for ch in range(nb // 16):
            s = pl.ds(ch * 16, 16)
            iidx[s] = _perm_idx(iidx[s])

        pltpu.async_copy(iemb_h.at[iidx], ibuf, semi)

        for j in range(nb // 16):
            lidx[pl.ds(j * 16, 16)] = lax.iota(jnp.int32, 16) + j * 16 + a0

        @pl.loop(0, nb)
        def _(i):
            for ch in range(H // 16):
                buf0[i, pl.ds(ch * 16, 16)] = jnp.zeros((16,), f32)

        pltpu.sync_copy(buf0, accum.at[pl.ds(a0, nb)])

        nring = 8
        bufs = (buf0, buf1, buf2, buf3, buf4, buf5, buf6, buf7)
        sems = (sem0, sem1, sem2, sem3, sem4, sem5, sem6, sem7)
        assert L % nring == 0 and L >= 2 * nring
        for k in range(nring):
            pltpu.async_copy(iemb_h.at[hist_v.at[k]], bufs[k], sems[k])

        @pl.loop(0, L - nring, step=nring)
        def _(l):
            for k in range(nring):
                pltpu.make_async_copy(
                    iemb_h.at[hist_v.at[l + k]], bufs[k], sems[k]).wait()
                pltpu.sync_copy(bufs[k], accum.at[lidx], add=True)
                pltpu.async_copy(
                    iemb_h.at[hist_v.at[l + nring + k]], bufs[k], sems[k])

        for k in range(nring):
            pltpu.make_async_copy(
                iemb_h.at[hist_v.at[L - nring + k]], bufs[k], sems[k]).wait()
            pltpu.sync_copy(bufs[k], accum.at[lidx], add=True)

        pltpu.make_async_copy(iemb_h.at[iidx], ibuf, semi).wait()
        pltpu.sync_copy(ibuf, it_out.at[pl.ds(b0, nb)])
        pltpu.sync_copy(accum.at[pl.ds(a0, nb)], cur_out.at[pl.ds(b0, nb)])

    return sc_b(item, hist, i_emb)


def _dice(x, alpha, eps=1e-8):
    mean = jnp.mean(x, axis=0, keepdims=True)
    var = jnp.mean((x - mean) ** 2, axis=0, keepdims=True)
    x_norm = (x - mean) / jnp.sqrt(var + eps)
    p = jax.nn.sigmoid(x_norm)
    return p * x + (1.0 - p) * alpha * x


def _mlp_body(u, it, c, cur, W1, b1, a1, W2, b2, a2, W3, b3, o):
    H = u.shape[1]
    f32 = jnp.float32
    x = jnp.dot(u[...], W1[0:H, :], preferred_element_type=f32)
    x = x + jnp.dot(it[...], W1[H:2 * H, :], preferred_element_type=f32)
    x = x + jnp.dot(c[...], W1[2 * H:3 * H, :], preferred_element_type=f32)
    x = x + jnp.dot(cur[...], W1[3 * H:4 * H, :], preferred_element_type=f32)
    x = x + b1[...]
    x = _dice(x, a1[...])
    x = jnp.dot(x, W2[...], preferred_element_type=f32) + b2[...]
    x = _dice(x, a2[...])
    o[...] = jnp.dot(x, W3[...], preferred_element_type=f32) + b3[...]


def kernel(user, hist, item, cate, u_emb, i_emb, c_emb, W1, b1, a1, W2, b2, a2, W3, b3):
    B = user.shape[0]
    u_emb = _detile(u_emb)
    c_emb = _detile(c_emb)
    u, c = _sc_small_gathers(user, cate, u_emb, c_emb)
    i_emb = _detile(i_emb)
    it, cur = _sc_pool(item, hist, i_emb)
    out = pl.pallas_call(
        _mlp_body,
        out_shape=jax.ShapeDtypeStruct((B, W3.shape[1]), jnp.float32),
    )(u, it, c, cur,
      W1, b1.reshape(1, -1), a1.reshape(1, -1),
      W2, b2.reshape(1, -1), a2.reshape(1, -1),
      W3, b3.reshape(1, -1))
    return out


# speedup vs baseline: 1.1922x; 1.0106x over previous
import functools

import jax
import jax.numpy as jnp
from jax import lax
from jax.experimental import pallas as pl
from jax.experimental.pallas import tpu as pltpu
from jax.experimental.pallas import tpu_sc as plsc

_NC = 2
_NS = 16
_NW = _NC * _NS
_DS = 16384

_MESH = plsc.VectorSubcoreMesh(core_axis_name="c", subcore_axis_name="s")
_CP = pltpu.CompilerParams(use_tc_tiling_on_sc=False)


def _perm_idx(v):
    s = _DS
    lg = s.bit_length() - 1
    return ((v & jnp.int32(-2 * s))
            | ((v & jnp.int32(s - 1)) << 1)
            | ((v >> lg) & jnp.int32(1)))


def _detile_body(t0, t1, tout):
    x = jnp.concatenate([t0[...], t1[...]], axis=0)
    tout[...] = jnp.transpose(x)


def _detile(t):
    V, H = t.shape
    S = _DS
    assert H == 64
    grid = (V + 2 * S - 1) // (2 * S)
    last = (V + S - 1) // S - 1
    out = pl.pallas_call(
        _detile_body,
        grid=(grid,),
        in_specs=[
            pl.BlockSpec((H, S), lambda j: (0, jnp.minimum(2 * j, last))),
            pl.BlockSpec((H, S), lambda j: (0, jnp.minimum(2 * j + 1, last))),
        ],
        out_specs=pl.BlockSpec((S, 2 * H), lambda j: (j, 0)),
        out_shape=jax.ShapeDtypeStruct((grid * S, 2 * H), jnp.float32),
    )(t.T, t.T)
    return out.reshape(grid * 2 * S, H)


def _sc_small_gathers(user, cate, u_emb, c_emb):
    B = user.shape[0]
    H = u_emb.shape[1]
    nb = B // _NW
    f32 = jnp.float32
    out_t = jax.ShapeDtypeStruct((B, H), f32)

    @functools.partial(
        pl.kernel,
        mesh=_MESH,
        out_type=[out_t, out_t],
        compiler_params=_CP,
        scratch_types=[
            pltpu.VMEM((nb, H), f32),
            pltpu.VMEM((nb, H), f32),
            pltpu.VMEM((nb,), jnp.int32),
            pltpu.VMEM((nb,), jnp.int32),
            pltpu.SemaphoreType.DMA,
            pltpu.SemaphoreType.DMA,
        ],
    )
    def sc_a(user_h, cate_h, uemb_h, cemb_h, u_out, c_out,
             ubuf, cbuf, uidx, cidx, semu, semc):
        wid = lax.axis_index("s") * _NC + lax.axis_index("c")
        b0 = wid * nb
        pltpu.sync_copy(user_h.at[pl.ds(b0, nb)], uidx)
        pltpu.sync_copy(cate_h.at[pl.ds(b0, nb)], cidx)
        for ch in range(nb // 16):
            s = pl.ds(ch * 16, 16)
            uidx[s] = _perm_idx(uidx[s])
            cidx[s] = _perm_idx(cidx[s])
        pltpu.async_copy(uemb_h.at[uidx], ubuf, semu)
        pltpu.async_copy(cemb_h.at[cidx], cbuf, semc)
        pltpu.make_async_copy(uemb_h.at[uidx], ubuf, semu).wait()
        pltpu.sync_copy(ubuf, u_out.at[pl.ds(b0, nb)])
        pltpu.make_async_copy(cemb_h.at[cidx], cbuf, semc).wait()
        pltpu.sync_copy(cbuf, c_out.at[pl.ds(b0, nb)])

    return sc_a(user, cate, u_emb, c_emb)


def _sc_pool(item, hist, i_emb):
    L, B = hist.shape
    H = i_emb.shape[1]
    nb = B // _NW
    assert B % (8 * _NW) == 0 and L % 2 == 0 and L >= 6 and nb <= 128
    assert H % 16 == 0
    f32 = jnp.float32
    out_t = jax.ShapeDtypeStruct((B, H), f32)

    @functools.partial(
        pl.kernel,
        mesh=_MESH,
        out_type=[out_t, out_t],
        compiler_params=_CP,
        scratch_types=[
            pltpu.VMEM((L, nb), jnp.int32),
            pltpu.VMEM_SHARED((B // _NC, H), f32),
            pltpu.VMEM((nb, H), f32),
            pltpu.VMEM((nb, H), f32),
            pltpu.VMEM((nb, H), f32),
            pltpu.VMEM((nb, H), f32),
            pltpu.VMEM((nb, H), f32),
            pltpu.VMEM((nb, H), f32),
            pltpu.VMEM((nb, H), f32),
            pltpu.VMEM((nb, H), f32),
            pltpu.VMEM((nb, H), f32),
            pltpu.VMEM((nb,), jnp.int32),
            pltpu.VMEM((nb,), jnp.int32),
            pltpu.SemaphoreType.DMA,
            pltpu.SemaphoreType.DMA,
            pltpu.SemaphoreType.DMA,
            pltpu.SemaphoreType.DMA,
            pltpu.SemaphoreType.DMA,
            pltpu.SemaphoreType.DMA,
            pltpu.SemaphoreType.DMA,
            pltpu.SemaphoreType.DMA,
            pltpu.SemaphoreType.DMA,
        ],
    )
    def sc_b(item_h, hist_h, iemb_h, it_out, cur_out,
             hist_v, accum, buf0, buf1, buf2, buf3, buf4, buf5, buf6, buf7,
             ibuf, iidx, lidx,
             sem0, sem1, sem2, sem3, sem4, sem5, sem6, sem7, semi):
        sid = lax.axis_index("s")
        wid = sid * _NC + lax.axis_index("c")
        b0 = wid * nb
        a0 = sid * nb

        pltpu.sync_copy(hist_h.at[:, pl.ds(b0, nb)], hist_v)
        pltpu.sync_copy(item_h.at[pl.ds(b0, nb)], iidx)

        @pl.loop(0, L)
        def _(l):
            for ch in range(nb // 16):
                s = pl.ds(ch * 16, 16)
                hist_v[l, s] = _perm_idx(hist_v[l, s])

        for ch in range(nb // 16):
            s = pl.ds(ch * 16, 16)
            iidx[s] = _perm_idx(iidx[s])

        pltpu.async_copy(iemb_h.at[iidx], ibuf, semi)

        for j in range(nb // 16):
            lidx[pl.ds(j * 16, 16)] = lax.iota(jnp.int32, 16) + j * 16 + a0

        @pl.loop(0, nb)
        def _(i):
            for ch in range(H // 16):
                buf0[i, pl.ds(ch * 16, 16)] = jnp.zeros((16,), f32)

        pltpu.sync_copy(buf0, accum.at[pl.ds(a0, nb)])

        nring = 8
        bufs = (buf0, buf1, buf2, buf3, buf4, buf5, buf6, buf7)
        sems = (sem0, sem1, sem2, sem3, sem4, sem5, sem6, sem7)
        assert L % nring == 0 and L >= 2 * nring
        for k in range(nring):
            pltpu.async_copy(iemb_h.at[hist_v.at[k]], bufs[k], sems[k])

        @pl.loop(0, L - nring, step=nring)
        def _(l):
            for k in range(nring):
                pltpu.make_async_copy(
                    iemb_h.at[hist_v.at[l + k]], bufs[k], sems[k]).wait()
                pltpu.sync_copy(bufs[k], accum.at[lidx], add=True)
                pltpu.async_copy(
                    iemb_h.at[hist_v.at[l + nring + k]], bufs[k], sems[k])

        for k in range(nring):
            pltpu.make_async_copy(
                iemb_h.at[hist_v.at[L - nring + k]], bufs[k], sems[k]).wait()
            pltpu.sync_copy(bufs[k], accum.at[lidx], add=True)

        pltpu.make_async_copy(iemb_h.at[iidx], ibuf, semi).wait()
        pltpu.sync_copy(ibuf, it_out.at[pl.ds(b0, nb)])
        pltpu.sync_copy(accum.at[pl.ds(a0, nb)], cur_out.at[pl.ds(b0, nb)])

    return sc_b(item, hist, i_emb)


def _dice(x, alpha, eps=1e-8):
    mean = jnp.mean(x, axis=0, keepdims=True)
    var = jnp.mean((x - mean) ** 2, axis=0, keepdims=True)
    x_norm = (x - mean) / jnp.sqrt(var + eps)
    p = jax.nn.sigmoid(x_norm)
    return p * x + (1.0 - p) * alpha * x


def _mlp_body(u, it, c, cur, W1, b1, a1, W2, b2, a2, W3, b3, o):
    H = u.shape[1]
    f32 = jnp.float32
    x = jnp.dot(u[...], W1[0:H, :], preferred_element_type=f32)
    x = x + jnp.dot(it[...], W1[H:2 * H, :], preferred_element_type=f32)
    x = x + jnp.dot(c[...], W1[2 * H:3 * H, :], preferred_element_type=f32)
    x = x + jnp.dot(cur[...], W1[3 * H:4 * H, :], preferred_element_type=f32)
    x = x + b1[...]
    x = _dice(x, a1[...])
    x = jnp.dot(x, W2[...], preferred_element_type=f32) + b2[...]
    x = _dice(x, a2[...])
    o[...] = jnp.dot(x, W3[...], preferred_element_type=f32) + b3[...]


def kernel(user, hist, item, cate, u_emb, i_emb, c_emb, W1, b1, a1, W2, b2, a2, W3, b3):
    B = user.shape[0]
    u_emb = _detile(u_emb)
    c_emb = _detile(c_emb)
    u, c = _sc_small_gathers(user, cate, u_emb, c_emb)
    i_emb = _detile(i_emb)
    it, cur = _sc_pool(item, hist, i_emb)
    out = pl.pallas_call(
        _mlp_body,
        out_shape=jax.ShapeDtypeStruct((B, W3.shape[1]), jnp.float32),
    )(u, it, c, cur,
      W1, b1.reshape(1, -1), a1.reshape(1, -1),
      W2, b2.reshape(1, -1), a2.reshape(1, -1),
      W3, b3.reshape(1, -1))
    return out
